# baseline (speedup 1.0000x reference)
import functools

import jax
import jax.numpy as jnp
from jax import lax
from jax.experimental import pallas as pl
from jax.experimental.pallas import tpu as pltpu
from jax.experimental.pallas import tpu_sc as plsc

N = 10000
E_RAW = 160000
E_TOT = E_RAW + N
E_PAD = 170496
EW = E_PAD // 16
EW32 = E_PAD // 32
NCH = 167
NCHP = 168
CHUNK = 64

_SC_MESH = dict(core_axis_name="c", subcore_axis_name="s",
                num_cores=2, num_subcores=16)
_SC_PARAMS = pltpu.CompilerParams(needs_layout_passes=False)


def _deg_body(dst_hbm, deg_out, dv, dpriv, red_v, out_v, stage):
    c = lax.axis_index("c")
    s = lax.axis_index("s")
    wid = c * 16 + s
    pltpu.sync_copy(dst_hbm.at[pl.ds(wid * EW32, EW32)], dv)

    zero16 = jnp.zeros((16,), jnp.float32)
    ones16 = jnp.ones((16,), jnp.float32)

    def zit(i, _):
        dpriv[pl.ds(i * 16, 16)] = zero16
        return 0
    lax.fori_loop(0, 640, zit, 0)

    def cit(i, _):
        d16 = dv[pl.ds(i * 16, 16)]
        plsc.addupdate_scatter(dpriv, [d16], ones16)
        return 0
    lax.fori_loop(0, EW32 // 16, cit, 0)

    pltpu.sync_copy(dpriv, stage.at[s])
    plsc.subcore_barrier()
    pltpu.sync_copy(stage.at[:, pl.ds(s * 640, 640)], red_v)

    def rit(j, _):
        acc = red_v[0, pl.ds(j * 16, 16)]
        for r in range(1, 16):
            acc = acc + red_v[r, pl.ds(j * 16, 16)]
        out_v[pl.ds(j * 16, 16)] = acc
        return 0
    lax.fori_loop(0, 40, rit, 0)

    pltpu.sync_copy(out_v, deg_out.at[c, pl.ds(s * 640, 640)])


def _deg_sc(dstf):
    return pl.kernel(
        _deg_body,
        out_type=jax.ShapeDtypeStruct((2, 10240), jnp.float32),
        mesh=plsc.VectorSubcoreMesh(**_SC_MESH),
        compiler_params=_SC_PARAMS,
        scratch_types=[
            pltpu.VMEM((EW32,), jnp.int32),
            pltpu.VMEM((10240,), jnp.float32),
            pltpu.VMEM((16, 640), jnp.float32),
            pltpu.VMEM((640,), jnp.float32),
            pltpu.VMEM_SHARED((16, 10240), jnp.float32),
        ],
    )(dstf)


def _bucket_body(src_hbm, dst_hbm, arena, cnt_hbm, sv, dv, ringf, cntv,
                 *, npass, rng):
    c = lax.axis_index("c")
    s = lax.axis_index("s")
    w = c * 16 + s
    base_node = c * 5120
    iota16 = lax.iota(jnp.int32, 16)
    zero16i = jnp.zeros((16,), jnp.int32)

    pltpu.sync_copy(src_hbm.at[pl.ds(s * EW, EW)], sv)
    pltpu.sync_copy(dst_hbm.at[pl.ds(s * EW, EW)], dv)

    def cit(i, cnts):
        s16 = sv[pl.ds(i * 16, 16)]
        d16 = dv[pl.ds(i * 16, 16)]
        out = []
        for p in range(npass):
            lo = base_node + p * rng
            m = jnp.logical_and(d16 >= lo, d16 < lo + rng)
            mi = m.astype(jnp.int32)
            pos = cnts[p] + plsc.cumsum(mi) - 1
            ridx = jax.lax.bitwise_and(pos, 127) + p * 128
            packed = jax.lax.bitwise_or(
                s16, jax.lax.shift_left(d16 - lo, 16))
            plsc.store_scatter(ringf, [ridx], packed, mask=m)
            ncnt = cnts[p] + jnp.sum(mi)
            oldch = jax.lax.shift_right_logical(cnts[p], 6)
            newch = jax.lax.shift_right_logical(ncnt, 6)

            @pl.when(newch > oldch)
            def _():
                roff = p * 128 + jax.lax.bitwise_and(oldch, 1) * 64
                pltpu.sync_copy(
                    ringf.at[pl.ds(roff, 64)],
                    arena.at[w, pl.ds((p * NCHP + oldch) * 64, 64)])
            out.append(ncnt)
        return tuple(out)

    cnts = lax.fori_loop(0, EW // 16, cit, (jnp.int32(0),) * npass)

    cv = zero16i
    for p in range(npass):
        lastch = jax.lax.shift_right_logical(cnts[p], 6)

        @pl.when(jax.lax.bitwise_and(cnts[p], 63) > 0)
        def _():
            roff = p * 128 + jax.lax.bitwise_and(lastch, 1) * 64
            pltpu.sync_copy(
                ringf.at[pl.ds(roff, 64)],
                arena.at[w, pl.ds((p * NCHP + lastch) * 64, 64)])
        cv = cv + cnts[p] * (iota16 == p).astype(jnp.int32)
    cntv[...] = cv
    pltpu.sync_copy(cntv, cnt_hbm.at[w])


def _bucket_sc(srcf, dstf, npass, rng):
    def body(src_hbm, dst_hbm, arena, cnt_hbm, *refs):
        _bucket_body(src_hbm, dst_hbm, arena, cnt_hbm, *refs,
                     npass=npass, rng=rng)

    return pl.kernel(
        body,
        out_type=(jax.ShapeDtypeStruct((32, npass * NCHP * CHUNK), jnp.int32),
                  jax.ShapeDtypeStruct((32, 16), jnp.int32)),
        mesh=plsc.VectorSubcoreMesh(**_SC_MESH),
        compiler_params=_SC_PARAMS,
        scratch_types=[
            pltpu.VMEM((EW,), jnp.int32),
            pltpu.VMEM((EW,), jnp.int32),
            pltpu.VMEM((npass * 128,), jnp.int32),
            pltpu.VMEM((16,), jnp.int32),
        ],
    )(srcf, dstf)


def _gather_body(table_hbm, arena, cnt_hbm, agg, cnt_all, chkbuf, pend,
                 sbuf, dbuf, rows, sbuf2, dbuf2, rows2, acc, sem, sem2,
                 *, C, npass, rng):
    c = lax.axis_index("c")
    s = lax.axis_index("s")
    base_node = c * 5120
    vpr = C // 16
    tr = rng // 16
    iota16 = lax.iota(jnp.int32, 16)
    zero16i = jnp.zeros((16,), jnp.int32)
    zero16f = jnp.zeros((16,), jnp.float32)

    pltpu.sync_copy(cnt_hbm, cnt_all)
    cols = [k * 16 + iota16 for k in range(vpr)]

    def prep_issue(base, n_edges, sbufx, dbufx, rowsx, semx):
        for g in range(4):
            v = pend[pl.ds(base + g * 16, 16)]
            valid = (g * 16 + iota16) < n_edges
            sbufx[pl.ds(g * 16, 16)] = jnp.where(
                valid, jax.lax.bitwise_and(v, 0xFFFF), 0)
            dbufx[pl.ds(g * 16, 16)] = jnp.where(
                valid, jax.lax.shift_right_logical(v, 16) - s * tr, tr)
        return pltpu.async_copy(table_hbm.at[sbufx], rowsx, semx)

    def consume(desc, dbufx, rowsx):
        desc.wait()

        @plsc.parallel_loop(0, 64)
        def eit(e):
            dspl = plsc.load_gather(dbufx, [zero16i + e])
            abase = dspl * C

            def kit(k2, _):
                for kk in range(4):
                    co = k2 * 64 + kk * 16
                    vals = rowsx[e, pl.ds(co, 16)]
                    plsc.addupdate_scatter(acc, [abase + co + iota16], vals)
                return 0
            lax.fori_loop(0, vpr // 4, kit, 0)

    def accumulate(base, n_edges):
        consume(prep_issue(base, n_edges, sbuf, dbuf, rows, sem),
                dbuf, rows)

    for p in range(npass):
        def zit(i, _):
            acc[pl.ds(i * 16, 16)] = zero16f
            return 0
        lax.fori_loop(0, (tr + 1) * vpr, zit, 0)

        lo_t = s * tr

        def per_subcore(u, wd):
            written, done = wd
            cnt_u = plsc.load_gather(cnt_all,
                                     [zero16i + (c * 16 + u), zero16i + p])
            cnt_up = jnp.minimum(jnp.maximum(jnp.max(cnt_u), 0), EW)

            def chunk_loop(jj, wd2):
                w2, d2 = wd2
                pltpu.sync_copy(
                    arena.at[c * 16 + u,
                             pl.ds((p * NCHP + jj * 8) * 64, 512)],
                    chkbuf)
                for g in range(32):
                    v = chkbuf[pl.ds(g * 16, 16)]
                    lane_g = jj * 512 + g * 16 + iota16
                    dloc = jax.lax.shift_right_logical(v, 16)
                    m = ((lane_g < cnt_up) & (dloc >= lo_t)
                         & (dloc < lo_t + tr))
                    mi = m.astype(jnp.int32)
                    pos = w2 + plsc.cumsum(mi) - 1
                    plsc.store_scatter(
                        pend, [jax.lax.bitwise_and(pos, 1023)], v, mask=m)
                    w2 = w2 + jnp.sum(mi)

                npair = jax.lax.shift_right_logical(w2 - d2, 7)

                def dit2(i, dd):
                    ba = jax.lax.bitwise_and(dd, 1023)
                    bb = jax.lax.bitwise_and(dd + 64, 1023)
                    da = prep_issue(ba, 64, sbuf, dbuf, rows, sem)
                    db = prep_issue(bb, 64, sbuf2, dbuf2, rows2, sem2)
                    consume(da, dbuf, rows)
                    consume(db, dbuf2, rows2)
                    return dd + 128
                d2 = lax.fori_loop(0, npair, dit2, d2)
                return (w2, d2)

            nsch_u = jax.lax.shift_right_logical(cnt_up + 511, 9)
            return lax.fori_loop(0, nsch_u, chunk_loop, (written, done))

        written, done = lax.fori_loop(
            0, 16, per_subcore, (jnp.int32(0), jnp.int32(0)))

        @pl.when(written - done >= 64)
        def _():
            accumulate(jax.lax.bitwise_and(done, 1023), 64)
        done = jnp.where(written - done >= 64, done + 64, done)

        @pl.when(written > done)
        def _():
            accumulate(jax.lax.bitwise_and(done, 1023), written - done)

        pltpu.sync_copy(
            acc.at[pl.ds(0, tr * C)],
            agg.at[pl.ds((base_node + p * rng + s * tr) * C, tr * C)])


def _gather_sc(table, arena, cnt, C, npass, rng):
    def body(table_hbm, arena_hbm, cnt_hbm, agg, *refs):
        _gather_body(table_hbm, arena_hbm, cnt_hbm, agg, *refs,
                     C=C, npass=npass, rng=rng)

    tr = rng // 16
    scratch = [
        pltpu.VMEM((32, 16), jnp.int32),
        pltpu.VMEM((512,), jnp.int32),
        pltpu.VMEM((1024,), jnp.int32),
        pltpu.VMEM((CHUNK,), jnp.int32),
        pltpu.VMEM((CHUNK,), jnp.int32),
        pltpu.VMEM((CHUNK, C), jnp.float32),
        pltpu.VMEM((CHUNK,), jnp.int32),
        pltpu.VMEM((CHUNK,), jnp.int32),
        pltpu.VMEM((CHUNK, C), jnp.float32),
        pltpu.VMEM(((tr + 1) * C,), jnp.float32),
        pltpu.SemaphoreType.DMA,
        pltpu.SemaphoreType.DMA,
    ]
    flat = pl.kernel(
        body,
        out_type=jax.ShapeDtypeStruct((10240 * C,), jnp.float32),
        mesh=plsc.VectorSubcoreMesh(**_SC_MESH),
        compiler_params=_SC_PARAMS,
        scratch_types=scratch,
    )(table, arena, cnt)
    return flat.reshape(10240, C)


def _agg_sc(srcf, dstf, table, C, npass, rng):
    arena, cnt = _bucket_sc(srcf, dstf, npass, rng)
    return _gather_sc(table, arena, cnt, C, npass, rng)


def _scale_body(dega_ref, degb_ref, x_ref, dis_ref, xs_ref):
    deg = dega_ref[...] + degb_ref[...]
    dis = lax.rsqrt(deg)
    dis_ref[...] = dis
    xs_ref[...] = x_ref[...] * dis


def _tc_scale(deg_a, deg_b, x):
    bm = 1000
    return pl.pallas_call(
        _scale_body,
        grid=(N // bm,),
        in_specs=[
            pl.BlockSpec((bm, 1), lambda i: (i, 0)),
            pl.BlockSpec((bm, 1), lambda i: (i, 0)),
            pl.BlockSpec((bm, 256), lambda i: (i, 0)),
        ],
        out_specs=[
            pl.BlockSpec((bm, 1), lambda i: (i, 0)),
            pl.BlockSpec((bm, 256), lambda i: (i, 0)),
        ],
        out_shape=[
            jax.ShapeDtypeStruct((N, 1), jnp.float32),
            jax.ShapeDtypeStruct((N, 256), jnp.float32),
        ],
    )(deg_a, deg_b, x)


def _mm1_body(agg_ref, dis_ref, w_ref, b_ref, o_ref):
    dis = dis_ref[...]
    h = jnp.dot(agg_ref[...] * dis, w_ref[...],
                preferred_element_type=jnp.float32)
    h = jnp.maximum(h + b_ref[...], 0.0)
    o_ref[...] = h * dis


def _tc_mm1(agg, dis, w_t, b):
    bm = 1000
    k, n = w_t.shape
    return pl.pallas_call(
        _mm1_body,
        grid=(N // bm,),
        in_specs=[
            pl.BlockSpec((bm, k), lambda i: (i, 0)),
            pl.BlockSpec((bm, 1), lambda i: (i, 0)),
            pl.BlockSpec((k, n), lambda i: (0, 0)),
            pl.BlockSpec((n,), lambda i: (0,)),
        ],
        out_specs=pl.BlockSpec((bm, n), lambda i: (i, 0)),
        out_shape=jax.ShapeDtypeStruct((N, n), jnp.float32),
    )(agg, dis, w_t, b)


def _mm2_body(agg_ref, dis_ref, w2_ref, b2_ref, wl_ref, bl_ref, o_ref):
    h = jnp.dot(agg_ref[...] * dis_ref[...], w2_ref[...],
                preferred_element_type=jnp.float32)
    h = jnp.maximum(h + b2_ref[...], 0.0)
    o_ref[...] = jnp.dot(h, wl_ref[...],
                         preferred_element_type=jnp.float32) + bl_ref[...]


def _tc_mm2(agg, dis, w2_t, b2, wl_t, bl):
    bm = 1000
    k, n = w2_t.shape
    ncls = wl_t.shape[1]
    return pl.pallas_call(
        _mm2_body,
        grid=(N // bm,),
        in_specs=[
            pl.BlockSpec((bm, k), lambda i: (i, 0)),
            pl.BlockSpec((bm, 1), lambda i: (i, 0)),
            pl.BlockSpec((k, n), lambda i: (0, 0)),
            pl.BlockSpec((n,), lambda i: (0,)),
            pl.BlockSpec((n, ncls), lambda i: (0, 0)),
            pl.BlockSpec((ncls,), lambda i: (0,)),
        ],
        out_specs=pl.BlockSpec((bm, ncls), lambda i: (i, 0)),
        out_shape=jax.ShapeDtypeStruct((N, ncls), jnp.float32),
    )(agg, dis, w2_t, b2, wl_t, bl)


def kernel(x, edge_index, W1, b1, W2, b2, Wl, bl):
    loop = jnp.arange(N, dtype=jnp.int32)
    pad = E_PAD - E_TOT
    srcf = jnp.concatenate([edge_index[0].astype(jnp.int32), loop,
                            jnp.zeros((pad,), jnp.int32)])
    dstf = jnp.concatenate([edge_index[1].astype(jnp.int32), loop,
                            jnp.full((pad,), N, jnp.int32)])

    deg_pp = _deg_sc(dstf)
    dis, xs = _tc_scale(deg_pp[0, :N, None], deg_pp[1, :N, None], x)

    agg1 = _agg_sc(srcf, dstf, xs, C=256, npass=2, rng=2560)
    h1s = _tc_mm1(agg1, dis, W1.T, b1)
    agg2 = _agg_sc(srcf, dstf, h1s, C=512, npass=4, rng=1280)
    return _tc_mm2(agg2, dis, W2.T, b2, Wl.T, bl)


# speedup vs baseline: 1.0911x; 1.0911x over previous
import functools

import jax
import jax.numpy as jnp
from jax import lax
from jax.experimental import pallas as pl
from jax.experimental.pallas import tpu as pltpu
from jax.experimental.pallas import tpu_sc as plsc

N = 10000
E_RAW = 160000
E_TOT = E_RAW + N
E_PAD = 170496
EW = E_PAD // 16
EW32 = E_PAD // 32
NCH = 167
NCHP = 168
CHUNK = 64

_SC_MESH = dict(core_axis_name="c", subcore_axis_name="s",
                num_cores=2, num_subcores=16)
_SC_PARAMS = pltpu.CompilerParams(needs_layout_passes=False)


def _deg_body(dst_hbm, deg_out, dv, dpriv, red_v, out_v, stage):
    c = lax.axis_index("c")
    s = lax.axis_index("s")
    wid = c * 16 + s
    pltpu.sync_copy(dst_hbm.at[pl.ds(wid * EW32, EW32)], dv)

    zero16 = jnp.zeros((16,), jnp.float32)
    ones16 = jnp.ones((16,), jnp.float32)

    def zit(i, _):
        dpriv[pl.ds(i * 16, 16)] = zero16
        return 0
    lax.fori_loop(0, 640, zit, 0)

    def cit(i, _):
        d16 = dv[pl.ds(i * 16, 16)]
        plsc.addupdate_scatter(dpriv, [d16], ones16)
        return 0
    lax.fori_loop(0, EW32 // 16, cit, 0)

    pltpu.sync_copy(dpriv, stage.at[s])
    plsc.subcore_barrier()
    pltpu.sync_copy(stage.at[:, pl.ds(s * 640, 640)], red_v)

    def rit(j, _):
        acc = red_v[0, pl.ds(j * 16, 16)]
        for r in range(1, 16):
            acc = acc + red_v[r, pl.ds(j * 16, 16)]
        out_v[pl.ds(j * 16, 16)] = acc
        return 0
    lax.fori_loop(0, 40, rit, 0)

    pltpu.sync_copy(out_v, deg_out.at[c, pl.ds(s * 640, 640)])


def _deg_sc(dstf):
    return pl.kernel(
        _deg_body,
        out_type=jax.ShapeDtypeStruct((2, 10240), jnp.float32),
        mesh=plsc.VectorSubcoreMesh(**_SC_MESH),
        compiler_params=_SC_PARAMS,
        scratch_types=[
            pltpu.VMEM((EW32,), jnp.int32),
            pltpu.VMEM((10240,), jnp.float32),
            pltpu.VMEM((16, 640), jnp.float32),
            pltpu.VMEM((640,), jnp.float32),
            pltpu.VMEM_SHARED((16, 10240), jnp.float32),
        ],
    )(dstf)


def _bucket_body(src_hbm, dst_hbm, arena, cnt_hbm, sv, dv, ringf, cntv,
                 *, npass, rng):
    c = lax.axis_index("c")
    s = lax.axis_index("s")
    w = c * 16 + s
    base_node = c * 5120
    iota16 = lax.iota(jnp.int32, 16)
    zero16i = jnp.zeros((16,), jnp.int32)

    pltpu.sync_copy(src_hbm.at[pl.ds(s * EW, EW)], sv)
    pltpu.sync_copy(dst_hbm.at[pl.ds(s * EW, EW)], dv)

    def cit(i, cnts):
        s16 = sv[pl.ds(i * 16, 16)]
        d16 = dv[pl.ds(i * 16, 16)]
        out = []
        for p in range(npass):
            lo = base_node + p * rng
            m = jnp.logical_and(d16 >= lo, d16 < lo + rng)
            mi = m.astype(jnp.int32)
            pos = cnts[p] + plsc.cumsum(mi) - 1
            ridx = jax.lax.bitwise_and(pos, 127) + p * 128
            packed = jax.lax.bitwise_or(
                s16, jax.lax.shift_left(d16 - lo, 16))
            plsc.store_scatter(ringf, [ridx], packed, mask=m)
            ncnt = cnts[p] + jnp.sum(mi)
            oldch = jax.lax.shift_right_logical(cnts[p], 6)
            newch = jax.lax.shift_right_logical(ncnt, 6)

            @pl.when(newch > oldch)
            def _():
                roff = p * 128 + jax.lax.bitwise_and(oldch, 1) * 64
                pltpu.sync_copy(
                    ringf.at[pl.ds(roff, 64)],
                    arena.at[w, pl.ds((p * NCHP + oldch) * 64, 64)])
            out.append(ncnt)
        return tuple(out)

    cnts = lax.fori_loop(0, EW // 16, cit, (jnp.int32(0),) * npass)

    cv = zero16i
    for p in range(npass):
        lastch = jax.lax.shift_right_logical(cnts[p], 6)

        @pl.when(jax.lax.bitwise_and(cnts[p], 63) > 0)
        def _():
            roff = p * 128 + jax.lax.bitwise_and(lastch, 1) * 64
            pltpu.sync_copy(
                ringf.at[pl.ds(roff, 64)],
                arena.at[w, pl.ds((p * NCHP + lastch) * 64, 64)])
        cv = cv + cnts[p] * (iota16 == p).astype(jnp.int32)
    cntv[...] = cv
    pltpu.sync_copy(cntv, cnt_hbm.at[w])


def _bucket_sc(srcf, dstf, npass, rng):
    def body(src_hbm, dst_hbm, arena, cnt_hbm, *refs):
        _bucket_body(src_hbm, dst_hbm, arena, cnt_hbm, *refs,
                     npass=npass, rng=rng)

    return pl.kernel(
        body,
        out_type=(jax.ShapeDtypeStruct((32, npass * NCHP * CHUNK), jnp.int32),
                  jax.ShapeDtypeStruct((32, 16), jnp.int32)),
        mesh=plsc.VectorSubcoreMesh(**_SC_MESH),
        compiler_params=_SC_PARAMS,
        scratch_types=[
            pltpu.VMEM((EW,), jnp.int32),
            pltpu.VMEM((EW,), jnp.int32),
            pltpu.VMEM((npass * 128,), jnp.int32),
            pltpu.VMEM((16,), jnp.int32),
        ],
    )(srcf, dstf)


def _gather_body(table_hbm, arena, cnt_hbm, agg, cnt_all, chkbuf, pend,
                 sbuf, dbuf, rows, sbuf2, dbuf2, rows2, acc, sem, sem2,
                 *, C, npass, rng):
    c = lax.axis_index("c")
    s = lax.axis_index("s")
    base_node = c * 5120
    vpr = C // 16
    tr = rng // 16
    iota16 = lax.iota(jnp.int32, 16)
    zero16i = jnp.zeros((16,), jnp.int32)
    zero16f = jnp.zeros((16,), jnp.float32)

    pltpu.sync_copy(cnt_hbm, cnt_all)
    cols = [k * 16 + iota16 for k in range(vpr)]

    def prep_issue(base, n_edges, sbufx, dbufx, rowsx, semx):
        for g in range(4):
            v = pend[pl.ds(base + g * 16, 16)]
            valid = (g * 16 + iota16) < n_edges
            sbufx[pl.ds(g * 16, 16)] = jnp.where(
                valid, jax.lax.bitwise_and(v, 0xFFFF), 0)
            dbufx[pl.ds(g * 16, 16)] = jnp.where(
                valid, jax.lax.shift_right_logical(v, 16) - s * tr, tr)
        return pltpu.async_copy(table_hbm.at[sbufx], rowsx, semx)

    def consume(desc, dbufx, rowsx):
        desc.wait()

        def eit(e, _):
            dspl = plsc.load_gather(dbufx, [zero16i + e])
            abase = dspl * C
            for k in range(vpr):
                vals = rowsx[e, pl.ds(k * 16, 16)]
                plsc.addupdate_scatter(acc, [abase + cols[k]], vals)
            return 0
        lax.fori_loop(0, 64, eit, 0)

    def accumulate(base, n_edges):
        consume(prep_issue(base, n_edges, sbuf, dbuf, rows, sem),
                dbuf, rows)

    for p in range(npass):
        def zit(i, _):
            acc[pl.ds(i * 16, 16)] = zero16f
            return 0
        lax.fori_loop(0, (tr + 1) * vpr, zit, 0)

        lo_t = s * tr

        def per_subcore(u, wd):
            written, done = wd
            cnt_u = plsc.load_gather(cnt_all,
                                     [zero16i + (c * 16 + u), zero16i + p])
            cnt_up = jnp.minimum(jnp.maximum(jnp.max(cnt_u), 0), EW)

            def chunk_loop(jj, wd2):
                w2, d2 = wd2
                pltpu.sync_copy(
                    arena.at[c * 16 + u,
                             pl.ds((p * NCHP + jj * 8) * 64, 512)],
                    chkbuf)
                for g in range(32):
                    v = chkbuf[pl.ds(g * 16, 16)]
                    lane_g = jj * 512 + g * 16 + iota16
                    dloc = jax.lax.shift_right_logical(v, 16)
                    m = ((lane_g < cnt_up) & (dloc >= lo_t)
                         & (dloc < lo_t + tr))
                    mi = m.astype(jnp.int32)
                    pos = w2 + plsc.cumsum(mi) - 1
                    plsc.store_scatter(
                        pend, [jax.lax.bitwise_and(pos, 1023)], v, mask=m)
                    w2 = w2 + jnp.sum(mi)

                npair = jax.lax.shift_right_logical(w2 - d2, 7)

                def dit2(i, dd):
                    ba = jax.lax.bitwise_and(dd, 1023)
                    bb = jax.lax.bitwise_and(dd + 64, 1023)
                    da = prep_issue(ba, 64, sbuf, dbuf, rows, sem)
                    db = prep_issue(bb, 64, sbuf2, dbuf2, rows2, sem2)
                    consume(da, dbuf, rows)
                    consume(db, dbuf2, rows2)
                    return dd + 128
                d2 = lax.fori_loop(0, npair, dit2, d2)
                return (w2, d2)

            nsch_u = jax.lax.shift_right_logical(cnt_up + 511, 9)
            return lax.fori_loop(0, nsch_u, chunk_loop, (written, done))

        written, done = lax.fori_loop(
            0, 16, per_subcore, (jnp.int32(0), jnp.int32(0)))

        @pl.when(written - done >= 64)
        def _():
            accumulate(jax.lax.bitwise_and(done, 1023), 64)
        done = jnp.where(written - done >= 64, done + 64, done)

        @pl.when(written > done)
        def _():
            accumulate(jax.lax.bitwise_and(done, 1023), written - done)

        pltpu.sync_copy(
            acc.at[pl.ds(0, tr * C)],
            agg.at[pl.ds((base_node + p * rng + s * tr) * C, tr * C)])


def _gather_sc(table, arena, cnt, C, npass, rng):
    def body(table_hbm, arena_hbm, cnt_hbm, agg, *refs):
        _gather_body(table_hbm, arena_hbm, cnt_hbm, agg, *refs,
                     C=C, npass=npass, rng=rng)

    tr = rng // 16
    scratch = [
        pltpu.VMEM((32, 16), jnp.int32),
        pltpu.VMEM((512,), jnp.int32),
        pltpu.VMEM((1024,), jnp.int32),
        pltpu.VMEM((CHUNK,), jnp.int32),
        pltpu.VMEM((CHUNK,), jnp.int32),
        pltpu.VMEM((CHUNK, C), jnp.float32),
        pltpu.VMEM((CHUNK,), jnp.int32),
        pltpu.VMEM((CHUNK,), jnp.int32),
        pltpu.VMEM((CHUNK, C), jnp.float32),
        pltpu.VMEM(((tr + 1) * C,), jnp.float32),
        pltpu.SemaphoreType.DMA,
        pltpu.SemaphoreType.DMA,
    ]
    flat = pl.kernel(
        body,
        out_type=jax.ShapeDtypeStruct((10240 * C,), jnp.float32),
        mesh=plsc.VectorSubcoreMesh(**_SC_MESH),
        compiler_params=_SC_PARAMS,
        scratch_types=scratch,
    )(table, arena, cnt)
    return flat.reshape(10240, C)


def _agg_sc(srcf, dstf, table, C, npass, rng):
    arena, cnt = _bucket_sc(srcf, dstf, npass, rng)
    return _gather_sc(table, arena, cnt, C, npass, rng)


def _scale_body(dega_ref, degb_ref, x_ref, dis_ref, xs_ref):
    deg = dega_ref[...] + degb_ref[...]
    dis = lax.rsqrt(deg)
    dis_ref[...] = dis
    xs_ref[...] = x_ref[...] * dis


def _tc_scale(deg_a, deg_b, x):
    bm = 1000
    return pl.pallas_call(
        _scale_body,
        grid=(N // bm,),
        in_specs=[
            pl.BlockSpec((bm, 1), lambda i: (i, 0)),
            pl.BlockSpec((bm, 1), lambda i: (i, 0)),
            pl.BlockSpec((bm, 256), lambda i: (i, 0)),
        ],
        out_specs=[
            pl.BlockSpec((bm, 1), lambda i: (i, 0)),
            pl.BlockSpec((bm, 256), lambda i: (i, 0)),
        ],
        out_shape=[
            jax.ShapeDtypeStruct((N, 1), jnp.float32),
            jax.ShapeDtypeStruct((N, 256), jnp.float32),
        ],
    )(deg_a, deg_b, x)


def _mm1_body(agg_ref, dis_ref, w_ref, b_ref, o_ref):
    dis = dis_ref[...]
    h = jnp.dot(agg_ref[...] * dis, w_ref[...],
                preferred_element_type=jnp.float32)
    h = jnp.maximum(h + b_ref[...], 0.0)
    o_ref[...] = h * dis


def _tc_mm1(agg, dis, w_t, b):
    bm = 1000
    k, n = w_t.shape
    return pl.pallas_call(
        _mm1_body,
        grid=(N // bm,),
        in_specs=[
            pl.BlockSpec((bm, k), lambda i: (i, 0)),
            pl.BlockSpec((bm, 1), lambda i: (i, 0)),
            pl.BlockSpec((k, n), lambda i: (0, 0)),
            pl.BlockSpec((n,), lambda i: (0,)),
        ],
        out_specs=pl.BlockSpec((bm, n), lambda i: (i, 0)),
        out_shape=jax.ShapeDtypeStruct((N, n), jnp.float32),
    )(agg, dis, w_t, b)


def _mm2_body(agg_ref, dis_ref, w2_ref, b2_ref, wl_ref, bl_ref, o_ref):
    h = jnp.dot(agg_ref[...] * dis_ref[...], w2_ref[...],
                preferred_element_type=jnp.float32)
    h = jnp.maximum(h + b2_ref[...], 0.0)
    o_ref[...] = jnp.dot(h, wl_ref[...],
                         preferred_element_type=jnp.float32) + bl_ref[...]


def _tc_mm2(agg, dis, w2_t, b2, wl_t, bl):
    bm = 1000
    k, n = w2_t.shape
    ncls = wl_t.shape[1]
    return pl.pallas_call(
        _mm2_body,
        grid=(N // bm,),
        in_specs=[
            pl.BlockSpec((bm, k), lambda i: (i, 0)),
            pl.BlockSpec((bm, 1), lambda i: (i, 0)),
            pl.BlockSpec((k, n), lambda i: (0, 0)),
            pl.BlockSpec((n,), lambda i: (0,)),
            pl.BlockSpec((n, ncls), lambda i: (0, 0)),
            pl.BlockSpec((ncls,), lambda i: (0,)),
        ],
        out_specs=pl.BlockSpec((bm, ncls), lambda i: (i, 0)),
        out_shape=jax.ShapeDtypeStruct((N, ncls), jnp.float32),
    )(agg, dis, w2_t, b2, wl_t, bl)


def kernel(x, edge_index, W1, b1, W2, b2, Wl, bl):
    loop = jnp.arange(N, dtype=jnp.int32)
    pad = E_PAD - E_TOT
    srcf = jnp.concatenate([edge_index[0].astype(jnp.int32), loop,
                            jnp.zeros((pad,), jnp.int32)])
    dstf = jnp.concatenate([edge_index[1].astype(jnp.int32), loop,
                            jnp.full((pad,), N, jnp.int32)])

    deg_pp = _deg_sc(dstf)
    dis, xs = _tc_scale(deg_pp[0, :N, None], deg_pp[1, :N, None], x)

    agg1 = _agg_sc(srcf, dstf, xs, C=256, npass=2, rng=2560)
    h1s = _tc_mm1(agg1, dis, W1.T, b1)
    agg2 = _agg_sc(srcf, dstf, h1s, C=512, npass=4, rng=1280)
    return _tc_mm2(agg2, dis, W2.T, b2, Wl.T, bl)
